# baseline (speedup 1.0000x reference)
import functools

import jax
import jax.numpy as jnp
from jax import lax
from jax.experimental import pallas as pl
from jax.experimental.pallas import tpu as pltpu
from jax.experimental.pallas import tpu_sc as plsc

EPS_ = 1e-05
K_ = 256
BS_ = 512
NPAD_ = 320


def _chunk_sum(q):
    b = q[:, 0:8]
    for k in range(1, 16):
        b = b + q[:, 8 * k:8 * k + 8]
    e0 = b[:, 0:1] + b[:, 4:5]
    e1 = b[:, 2:3] + b[:, 6:7]
    e2 = b[:, 1:2] + b[:, 5:6]
    e3 = b[:, 3:4] + b[:, 7:8]
    return (e0 + e1) + (e2 + e3)


def _norm2(x):
    s = x * x
    acc = None
    for j in range(3):
        q = s[:, 256 * j:256 * j + 128] + s[:, 256 * j + 128:256 * j + 256]
        c = _chunk_sum(q)
        acc = c if acc is None else acc + c
    return acc


def _rowdot(p):
    acc = None
    for c in range(6):
        sc = _chunk_sum(p[:, 128 * c:128 * c + 128])
        acc = sc if acc is None else acc + sc
    return acc


def _tile_sum(xt):
    b = xt[0:8, :]
    for k in range(1, 16):
        b = b + xt[8 * k:8 * k + 8, :]
    e0 = b[0:1, :] + b[4:5, :]
    e1 = b[2:3, :] + b[6:7, :]
    e2 = b[1:2, :] + b[5:6, :]
    e3 = b[3:4, :] + b[7:8, :]
    return (e0 + e1) + (e2 + e3)


def _norm2_lanes(x):
    s = x * x
    q = [s[:, 256 * j:256 * j + 128] + s[:, 256 * j + 128:256 * j + 256]
         for j in range(3)]
    groups = []
    for g in range(x.shape[0] // 128):
        acc = None
        for j in range(3):
            c = _tile_sum(q[j][128 * g:128 * g + 128, :].T)
            acc = c if acc is None else acc + c
        groups.append(acc)
    return jnp.concatenate(groups, axis=0)


def _rowdot_lanes(p):
    groups = []
    for g in range(p.shape[0] // 128):
        acc = None
        for c in range(6):
            sc = _tile_sum(p[128 * g:128 * g + 128, 128 * c:128 * c + 128].T)
            acc = sc if acc is None else acc + sc
        groups.append(acc)
    return jnp.concatenate(groups, axis=0)


def _sims_body(a_ref, b_ref, o_ref):
    a = a_ref[0]
    b0 = b_ref[0, 0:1]
    n2l = _norm2_lanes(a)
    an = jnp.sqrt(n2l) + EPS_
    an_col = jnp.concatenate(
        [an[g:g + 1, :].T for g in range(BS_ // 128)], axis=0)
    av = a / an_col
    bn = jnp.sqrt(_norm2(b0)) + EPS_
    bv = b0 / bn
    nxt_v = jnp.concatenate([av[1:], bv], axis=0)
    dl = _rowdot_lanes(av * nxt_v)
    s = jnp.abs(dl)
    for g in range(BS_ // 128):
        o_ref[0, 0, 128 * g:128 * (g + 1)] = s[g, :]


def _similarities(video):
    B, T, D = video.shape
    nj = T // BS_
    out = pl.pallas_call(
        _sims_body,
        grid=(B, nj),
        in_specs=[
            pl.BlockSpec((1, BS_, D), lambda b, j: (b, j, 0)),
            pl.BlockSpec((1, 8, D),
                         lambda b, j: (b, jnp.minimum(j + 1, nj - 1) * (BS_ // 8), 0)),
        ],
        out_specs=pl.BlockSpec((1, 1, BS_), lambda b, j: (b * nj + j, 0, 0)),
        out_shape=jax.ShapeDtypeStruct((B * nj, 1, BS_), jnp.float32),
    )(video, video)
    return out.reshape(B, T)


def _splat(x, val, dtype=jnp.int32):
    del x
    return jnp.full((16,), val, dtype)


def _scalar(v16):
    return lax.reduce_max(v16, axes=(0,))


def _make_topk_gather(B, T, D, DA):
    mesh = plsc.VectorSubcoreMesh(core_axis_name="c", subcore_axis_name="s")

    @functools.partial(
        pl.kernel,
        out_type=[jax.ShapeDtypeStruct((B * NPAD_, D), jnp.float32),
                  jax.ShapeDtypeStruct((B * NPAD_, DA), jnp.float32)],
        mesh=mesh,
        compiler_params=pltpu.CompilerParams(needs_layout_passes=False),
        scratch_types=[
            pltpu.VMEM((T,), jnp.int32),
            pltpu.VMEM((4096,), jnp.int32),
            pltpu.VMEM((256,), jnp.int32),
            pltpu.VMEM((272,), jnp.int32),
            pltpu.VMEM((272,), jnp.int32),
            pltpu.VMEM((T + 16,), jnp.int32),
            pltpu.VMEM((NPAD_,), jnp.int32),
            pltpu.VMEM_SHARED((2 * NPAD_,), jnp.int32),
            pltpu.VMEM((40,), jnp.int32),
            pltpu.VMEM((40, D), jnp.float32),
            pltpu.VMEM((40, DA), jnp.float32),
            pltpu.SemaphoreType.DMA,
        ],
    )
    def topk_gather(sims_hbm, vid_hbm, aud_hbm, vout_hbm, aout_hbm,
                    bits_v, hist_v, cum_v, candv_v, candp_v, eq_v, oidx_v,
                    shared, gidx_v, vrows_v, arows_v, sem):
        c = lax.axis_index("c")
        s = lax.axis_index("s")
        row = 2 * c + s
        lane = lax.broadcasted_iota(jnp.int32, (16,), 0)
        ones16 = jnp.full((16,), 1, jnp.int32)
        zeros16 = jnp.zeros((16,), jnp.int32)

        @pl.when(s < 2)
        def _phase_a():
            pltpu.sync_copy(sims_hbm.at[pl.ds(row * T, T)], bits_v)

            def one_pass(shift, prefix, base, first):
                def zero_body(i, _):
                    hist_v[pl.ds(i * 16, 16)] = zeros16
                    return 0
                lax.fori_loop(0, 256, zero_body, 0)

                def scan_body(i, _):
                    v = bits_v[pl.ds(i * 16, 16)]
                    d = lax.shift_right_logical(v, shift) & 0xFF
                    if first:
                        m = None
                    else:
                        m = lax.shift_right_logical(v, shift + 8) == prefix
                    plsc.addupdate_scatter(hist_v, [lane * 256 + d], ones16,
                                           mask=m)
                    return 0
                lax.fori_loop(0, T // 16, scan_body, 0)

                def comb_body(i, carry):
                    acc = zeros16
                    for l in range(16):
                        acc = acc + hist_v[pl.ds(l * 256 + i * 16, 16)]
                    cum = plsc.cumsum(acc) + carry
                    cum_v[pl.ds(i * 16, 16)] = cum
                    return carry + jnp.sum(acc)
                lax.fori_loop(0, 16, comb_body, jnp.int32(0))

                def find_body(i, acc):
                    cum = cum_v[pl.ds(i * 16, 16)]
                    below = (base + cum) < K_
                    return acc + jnp.sum(jnp.where(below, 1, 0))
                bucket = lax.fori_loop(0, 16, find_body, jnp.int32(0))

                prev = _scalar(plsc.load_gather(
                    cum_v, [_splat(None, jnp.maximum(bucket - 1, 0))]))
                base2 = base + jnp.where(bucket == 0, 0, prev)
                if first:
                    prefix2 = bucket
                else:
                    prefix2 = (prefix << 8) | bucket
                return prefix2, base2

            prefix, base = one_pass(24, jnp.int32(0), jnp.int32(0), True)
            prefix, base = one_pass(16, prefix, base, False)
            prefix, base = one_pass(8, prefix, base, False)
            kstar, base = one_pass(0, prefix, base, False)
            need = K_ - base

            def coll_body(i, carry):
                off_lt, off_eq = carry
                v = bits_v[pl.ds(i * 16, 16)]
                pos = lane + i * 16
                mlt = v < kstar
                meq = v == kstar
                ilt = jnp.where(mlt, 1, 0)
                ieq = jnp.where(meq, 1, 0)
                dst_lt = off_lt + plsc.cumsum(ilt) - ilt
                dst_eq = off_eq + plsc.cumsum(ieq) - ieq
                plsc.store_scatter(candv_v, [dst_lt], v, mask=mlt)
                plsc.store_scatter(candp_v, [dst_lt], pos, mask=mlt)
                plsc.store_scatter(eq_v, [dst_eq], pos, mask=meq)
                nlt = jnp.sum(ilt)
                neq = jnp.sum(ieq)
                return off_lt + nlt, off_eq + neq
            lax.fori_loop(0, T // 16, coll_body,
                          (jnp.int32(0), jnp.int32(0)))

            def app_body(j, _):
                @pl.when(j * 16 < need)
                def _():
                    candv_v[pl.ds(base + j * 16, 16)] = jnp.full(
                        (16,), kstar, jnp.int32)
                    candp_v[pl.ds(base + j * 16, 16)] = eq_v[pl.ds(j * 16, 16)]
                return 0
            lax.fori_loop(0, 16, app_body, 0)

            def pad_body(m, _):
                oidx_v[pl.ds(m * 16, 16)] = jnp.full((16,), row * T,
                                                     jnp.int32)
                return 0
            lax.fori_loop(0, NPAD_ // 16, pad_body, 0)

            def rank_body(e, _):
                ev = plsc.load_gather(candv_v, [_splat(None, e)])
                ep = plsc.load_gather(candp_v, [_splat(None, e)])
                def inner(i, acc):
                    v = candv_v[pl.ds(i * 16, 16)]
                    p = candp_v[pl.ds(i * 16, 16)]
                    lt = (v < ev) | ((v == ev) & (p < ep))
                    return acc + jnp.where(lt, 1, 0)
                rank = jnp.sum(lax.fori_loop(0, 16, inner, zeros16))
                t = _scalar(ep)
                plsc.store_scatter(oidx_v, [_splat(None, rank + 1)],
                                   _splat(None, row * T + t + 1),
                                   mask=lane == 0)
                return 0
            lax.fori_loop(0, K_, rank_body, 0)

            pltpu.sync_copy(oidx_v, shared.at[pl.ds(s * NPAD_, NPAD_)])

        plsc.subcore_barrier()

        br = s // 8
        kk = s % 8
        batch = 2 * c + br
        pltpu.sync_copy(shared.at[pl.ds(br * NPAD_ + 40 * kk, 40)], gidx_v)
        pltpu.async_copy(vid_hbm.at[gidx_v], vrows_v, sem).wait()
        pltpu.sync_copy(vrows_v,
                        vout_hbm.at[pl.ds(batch * NPAD_ + 40 * kk, 40)])
        pltpu.async_copy(aud_hbm.at[gidx_v], arows_v, sem).wait()
        pltpu.sync_copy(arows_v,
                        aout_hbm.at[pl.ds(batch * NPAD_ + 40 * kk, 40)])

    return topk_gather


def kernel(video, audio):
    B, T, D = video.shape
    DA = audio.shape[2]
    sims = _similarities(video)
    sims = sims.at[:, T - 1].set(jnp.inf)
    bits = lax.bitcast_convert_type(sims, jnp.int32)
    vout, aout = _make_topk_gather(B, T, D, DA)(
        bits.reshape(B * T), video.reshape(B * T, D),
        audio.reshape(B * T, DA))
    result_video = vout.reshape(B, NPAD_, D)[:, :K_ + 1]
    result_audio = aout.reshape(B, NPAD_, DA)[:, :K_ + 1]
    return (result_video, result_audio)


# speedup vs baseline: 1.1753x; 1.1753x over previous
import functools

import jax
import jax.numpy as jnp
from jax import lax
from jax.experimental import pallas as pl
from jax.experimental.pallas import tpu as pltpu
from jax.experimental.pallas import tpu_sc as plsc

EPS_ = 1e-05
K_ = 256
BS_ = 1024
NPAD_ = 320


def _chunk_sum(q):
    b = q[:, 0:8]
    for k in range(1, 16):
        b = b + q[:, 8 * k:8 * k + 8]
    e0 = b[:, 0:1] + b[:, 4:5]
    e1 = b[:, 2:3] + b[:, 6:7]
    e2 = b[:, 1:2] + b[:, 5:6]
    e3 = b[:, 3:4] + b[:, 7:8]
    return (e0 + e1) + (e2 + e3)


def _norm2(x):
    s = x * x
    acc = None
    for j in range(3):
        q = s[:, 256 * j:256 * j + 128] + s[:, 256 * j + 128:256 * j + 256]
        c = _chunk_sum(q)
        acc = c if acc is None else acc + c
    return acc


def _rowdot(p):
    acc = None
    for c in range(6):
        sc = _chunk_sum(p[:, 128 * c:128 * c + 128])
        acc = sc if acc is None else acc + sc
    return acc


def _tile_sum(xt):
    b = xt[0:8, :]
    for k in range(1, 16):
        b = b + xt[8 * k:8 * k + 8, :]
    e0 = b[0:1, :] + b[4:5, :]
    e1 = b[2:3, :] + b[6:7, :]
    e2 = b[1:2, :] + b[5:6, :]
    e3 = b[3:4, :] + b[7:8, :]
    return (e0 + e1) + (e2 + e3)


def _norm2_lanes(x):
    s = x * x
    q = [s[:, 256 * j:256 * j + 128] + s[:, 256 * j + 128:256 * j + 256]
         for j in range(3)]
    groups = []
    for g in range(x.shape[0] // 128):
        acc = None
        for j in range(3):
            c = _tile_sum(q[j][128 * g:128 * g + 128, :].T)
            acc = c if acc is None else acc + c
        groups.append(acc)
    return jnp.concatenate(groups, axis=0)


def _rowdot_lanes(p):
    groups = []
    for g in range(p.shape[0] // 128):
        acc = None
        for c in range(6):
            sc = _tile_sum(p[128 * g:128 * g + 128, 128 * c:128 * c + 128].T)
            acc = sc if acc is None else acc + sc
        groups.append(acc)
    return jnp.concatenate(groups, axis=0)


def _sims_body(a_ref, b_ref, o_ref):
    a = a_ref[0]
    b0 = b_ref[0, 0:1]
    n2l = _norm2_lanes(a)
    an = jnp.sqrt(n2l) + EPS_
    an_col = jnp.concatenate(
        [an[g:g + 1, :].T for g in range(BS_ // 128)], axis=0)
    av = a / an_col
    bn = jnp.sqrt(_norm2(b0)) + EPS_
    bv = b0 / bn
    nxt_v = jnp.concatenate([av[1:], bv], axis=0)
    dl = _rowdot_lanes(av * nxt_v)
    s = jnp.abs(dl)
    for g in range(BS_ // 128):
        o_ref[0, 0, 128 * g:128 * (g + 1)] = s[g, :]


def _similarities(video):
    B, T, D = video.shape
    nj = T // BS_
    out = pl.pallas_call(
        _sims_body,
        grid=(B, nj),
        in_specs=[
            pl.BlockSpec((1, BS_, D), lambda b, j: (b, j, 0)),
            pl.BlockSpec((1, 8, D),
                         lambda b, j: (b, jnp.minimum(j + 1, nj - 1) * (BS_ // 8), 0)),
        ],
        out_specs=pl.BlockSpec((1, 1, BS_), lambda b, j: (b * nj + j, 0, 0)),
        out_shape=jax.ShapeDtypeStruct((B * nj, 1, BS_), jnp.float32),
    )(video, video)
    return out.reshape(B, T)


def _splat(x, val, dtype=jnp.int32):
    del x
    return jnp.full((16,), val, dtype)


def _scalar(v16):
    return lax.reduce_max(v16, axes=(0,))


def _make_topk_gather(B, T, D, DA):
    mesh = plsc.VectorSubcoreMesh(core_axis_name="c", subcore_axis_name="s")

    @functools.partial(
        pl.kernel,
        out_type=[jax.ShapeDtypeStruct((B * NPAD_, D), jnp.float32),
                  jax.ShapeDtypeStruct((B * NPAD_, DA), jnp.float32)],
        mesh=mesh,
        compiler_params=pltpu.CompilerParams(needs_layout_passes=False),
        scratch_types=[
            pltpu.VMEM((T,), jnp.int32),
            pltpu.VMEM((4096,), jnp.int32),
            pltpu.VMEM((256,), jnp.int32),
            pltpu.VMEM((272,), jnp.int32),
            pltpu.VMEM((272,), jnp.int32),
            pltpu.VMEM((T + 16,), jnp.int32),
            pltpu.VMEM((NPAD_,), jnp.int32),
            pltpu.VMEM_SHARED((2 * NPAD_,), jnp.int32),
            pltpu.VMEM((40,), jnp.int32),
            pltpu.VMEM((40, D), jnp.float32),
            pltpu.VMEM((40, DA), jnp.float32),
            pltpu.SemaphoreType.DMA,
        ],
    )
    def topk_gather(sims_hbm, vid_hbm, aud_hbm, vout_hbm, aout_hbm,
                    bits_v, hist_v, cum_v, candv_v, candp_v, eq_v, oidx_v,
                    shared, gidx_v, vrows_v, arows_v, sem):
        c = lax.axis_index("c")
        s = lax.axis_index("s")
        row = 2 * c + s
        lane = lax.broadcasted_iota(jnp.int32, (16,), 0)
        ones16 = jnp.full((16,), 1, jnp.int32)
        zeros16 = jnp.zeros((16,), jnp.int32)

        @pl.when(s < 2)
        def _phase_a():
            pltpu.sync_copy(sims_hbm.at[pl.ds(row * T, T)], bits_v)

            def one_pass(shift, prefix, base, first):
                def zero_body(i, _):
                    hist_v[pl.ds(i * 16, 16)] = zeros16
                    return 0
                lax.fori_loop(0, 256, zero_body, 0)

                def scan_body(i, _):
                    v = bits_v[pl.ds(i * 16, 16)]
                    d = lax.shift_right_logical(v, shift) & 0xFF
                    if first:
                        m = None
                    else:
                        m = lax.shift_right_logical(v, shift + 8) == prefix
                    plsc.addupdate_scatter(hist_v, [lane * 256 + d], ones16,
                                           mask=m)
                    return 0
                lax.fori_loop(0, T // 16, scan_body, 0)

                def comb_body(i, carry):
                    acc = zeros16
                    for l in range(16):
                        acc = acc + hist_v[pl.ds(l * 256 + i * 16, 16)]
                    cum = plsc.cumsum(acc) + carry
                    cum_v[pl.ds(i * 16, 16)] = cum
                    return carry + jnp.sum(acc)
                lax.fori_loop(0, 16, comb_body, jnp.int32(0))

                def find_body(i, acc):
                    cum = cum_v[pl.ds(i * 16, 16)]
                    below = (base + cum) < K_
                    return acc + jnp.sum(jnp.where(below, 1, 0))
                bucket = lax.fori_loop(0, 16, find_body, jnp.int32(0))

                prev = _scalar(plsc.load_gather(
                    cum_v, [_splat(None, jnp.maximum(bucket - 1, 0))]))
                base2 = base + jnp.where(bucket == 0, 0, prev)
                if first:
                    prefix2 = bucket
                else:
                    prefix2 = (prefix << 8) | bucket
                return prefix2, base2

            prefix, base = one_pass(24, jnp.int32(0), jnp.int32(0), True)
            prefix, base = one_pass(16, prefix, base, False)
            prefix, base = one_pass(8, prefix, base, False)
            kstar, base = one_pass(0, prefix, base, False)
            need = K_ - base

            def coll_body(i, carry):
                off_lt, off_eq = carry
                v = bits_v[pl.ds(i * 16, 16)]
                pos = lane + i * 16
                mlt = v < kstar
                meq = v == kstar
                ilt = jnp.where(mlt, 1, 0)
                ieq = jnp.where(meq, 1, 0)
                dst_lt = off_lt + plsc.cumsum(ilt) - ilt
                dst_eq = off_eq + plsc.cumsum(ieq) - ieq
                plsc.store_scatter(candv_v, [dst_lt], v, mask=mlt)
                plsc.store_scatter(candp_v, [dst_lt], pos, mask=mlt)
                plsc.store_scatter(eq_v, [dst_eq], pos, mask=meq)
                nlt = jnp.sum(ilt)
                neq = jnp.sum(ieq)
                return off_lt + nlt, off_eq + neq
            lax.fori_loop(0, T // 16, coll_body,
                          (jnp.int32(0), jnp.int32(0)))

            def app_body(j, _):
                @pl.when(j * 16 < need)
                def _():
                    candv_v[pl.ds(base + j * 16, 16)] = jnp.full(
                        (16,), kstar, jnp.int32)
                    candp_v[pl.ds(base + j * 16, 16)] = eq_v[pl.ds(j * 16, 16)]
                return 0
            lax.fori_loop(0, 16, app_body, 0)

            def pad_body(m, _):
                oidx_v[pl.ds(m * 16, 16)] = jnp.full((16,), row * T,
                                                     jnp.int32)
                return 0
            lax.fori_loop(0, NPAD_ // 16, pad_body, 0)

            def rank_body(e, _):
                ev = plsc.load_gather(candv_v, [_splat(None, e)])
                ep = plsc.load_gather(candp_v, [_splat(None, e)])
                def inner(i, acc):
                    v = candv_v[pl.ds(i * 16, 16)]
                    p = candp_v[pl.ds(i * 16, 16)]
                    lt = (v < ev) | ((v == ev) & (p < ep))
                    return acc + jnp.where(lt, 1, 0)
                rank = jnp.sum(lax.fori_loop(0, 16, inner, zeros16))
                t = _scalar(ep)
                plsc.store_scatter(oidx_v, [_splat(None, rank + 1)],
                                   _splat(None, row * T + t + 1),
                                   mask=lane == 0)
                return 0
            lax.fori_loop(0, K_, rank_body, 0)

            pltpu.sync_copy(oidx_v, shared.at[pl.ds(s * NPAD_, NPAD_)])

        plsc.subcore_barrier()

        br = s // 8
        kk = s % 8
        batch = 2 * c + br
        pltpu.sync_copy(shared.at[pl.ds(br * NPAD_ + 40 * kk, 40)], gidx_v)
        pltpu.async_copy(vid_hbm.at[gidx_v], vrows_v, sem).wait()
        pltpu.sync_copy(vrows_v,
                        vout_hbm.at[pl.ds(batch * NPAD_ + 40 * kk, 40)])
        pltpu.async_copy(aud_hbm.at[gidx_v], arows_v, sem).wait()
        pltpu.sync_copy(arows_v,
                        aout_hbm.at[pl.ds(batch * NPAD_ + 40 * kk, 40)])

    return topk_gather


def kernel(video, audio):
    B, T, D = video.shape
    DA = audio.shape[2]
    sims = _similarities(video)
    sims = sims.at[:, T - 1].set(jnp.inf)
    bits = lax.bitcast_convert_type(sims, jnp.int32)
    vout, aout = _make_topk_gather(B, T, D, DA)(
        bits.reshape(B * T), video.reshape(B * T, D),
        audio.reshape(B * T, DA))
    result_video = vout.reshape(B, NPAD_, D)[:, :K_ + 1]
    result_audio = aout.reshape(B, NPAD_, DA)[:, :K_ + 1]
    return (result_video, result_audio)


# speedup vs baseline: 1.3616x; 1.1585x over previous
import functools

import jax
import jax.numpy as jnp
from jax import lax
from jax.experimental import pallas as pl
from jax.experimental.pallas import tpu as pltpu
from jax.experimental.pallas import tpu_sc as plsc

EPS_ = 1e-05
K_ = 256
BS_ = 2048
NPAD_ = 320


def _chunk_sum(q):
    b = q[:, 0:8]
    for k in range(1, 16):
        b = b + q[:, 8 * k:8 * k + 8]
    e0 = b[:, 0:1] + b[:, 4:5]
    e1 = b[:, 2:3] + b[:, 6:7]
    e2 = b[:, 1:2] + b[:, 5:6]
    e3 = b[:, 3:4] + b[:, 7:8]
    return (e0 + e1) + (e2 + e3)


def _norm2(x):
    s = x * x
    acc = None
    for j in range(3):
        q = s[:, 256 * j:256 * j + 128] + s[:, 256 * j + 128:256 * j + 256]
        c = _chunk_sum(q)
        acc = c if acc is None else acc + c
    return acc


def _rowdot(p):
    acc = None
    for c in range(6):
        sc = _chunk_sum(p[:, 128 * c:128 * c + 128])
        acc = sc if acc is None else acc + sc
    return acc


def _tile_sum(xt):
    b = xt[0:8, :]
    for k in range(1, 16):
        b = b + xt[8 * k:8 * k + 8, :]
    e0 = b[0:1, :] + b[4:5, :]
    e1 = b[2:3, :] + b[6:7, :]
    e2 = b[1:2, :] + b[5:6, :]
    e3 = b[3:4, :] + b[7:8, :]
    return (e0 + e1) + (e2 + e3)


def _norm2_lanes(x):
    s = x * x
    q = [s[:, 256 * j:256 * j + 128] + s[:, 256 * j + 128:256 * j + 256]
         for j in range(3)]
    groups = []
    for g in range(x.shape[0] // 128):
        acc = None
        for j in range(3):
            c = _tile_sum(q[j][128 * g:128 * g + 128, :].T)
            acc = c if acc is None else acc + c
        groups.append(acc)
    return jnp.concatenate(groups, axis=0)


def _rowdot_lanes(p):
    groups = []
    for g in range(p.shape[0] // 128):
        acc = None
        for c in range(6):
            sc = _tile_sum(p[128 * g:128 * g + 128, 128 * c:128 * c + 128].T)
            acc = sc if acc is None else acc + sc
        groups.append(acc)
    return jnp.concatenate(groups, axis=0)


def _sims_body(a_ref, b_ref, o_ref):
    a = a_ref[0]
    b0 = b_ref[0, 0:1]
    n2l = _norm2_lanes(a)
    an = jnp.sqrt(n2l) + EPS_
    an_col = jnp.concatenate(
        [an[g:g + 1, :].T for g in range(BS_ // 128)], axis=0)
    av = a / an_col
    bn = jnp.sqrt(_norm2(b0)) + EPS_
    bv = b0 / bn
    nxt_v = jnp.concatenate([av[1:], bv], axis=0)
    dl = _rowdot_lanes(av * nxt_v)
    s = jnp.abs(dl)
    for g in range(BS_ // 128):
        o_ref[0, 0, 128 * g:128 * (g + 1)] = s[g, :]


def _similarities(video):
    B, T, D = video.shape
    nj = T // BS_
    out = pl.pallas_call(
        _sims_body,
        grid=(B, nj),
        in_specs=[
            pl.BlockSpec((1, BS_, D), lambda b, j: (b, j, 0)),
            pl.BlockSpec((1, 8, D),
                         lambda b, j: (b, jnp.minimum(j + 1, nj - 1) * (BS_ // 8), 0)),
        ],
        out_specs=pl.BlockSpec((1, 1, BS_), lambda b, j: (b * nj + j, 0, 0)),
        out_shape=jax.ShapeDtypeStruct((B * nj, 1, BS_), jnp.float32),
    )(video, video)
    return out.reshape(B, T)


def _splat(x, val, dtype=jnp.int32):
    del x
    return jnp.full((16,), val, dtype)


def _scalar(v16):
    return lax.reduce_max(v16, axes=(0,))


def _make_topk_gather(B, T, D, DA):
    mesh = plsc.VectorSubcoreMesh(core_axis_name="c", subcore_axis_name="s")

    @functools.partial(
        pl.kernel,
        out_type=[jax.ShapeDtypeStruct((B * NPAD_, D), jnp.float32),
                  jax.ShapeDtypeStruct((B * NPAD_, DA), jnp.float32)],
        mesh=mesh,
        compiler_params=pltpu.CompilerParams(needs_layout_passes=False),
        scratch_types=[
            pltpu.VMEM((T,), jnp.int32),
            pltpu.VMEM((4096,), jnp.int32),
            pltpu.VMEM((256,), jnp.int32),
            pltpu.VMEM((272,), jnp.int32),
            pltpu.VMEM((272,), jnp.int32),
            pltpu.VMEM((T + 16,), jnp.int32),
            pltpu.VMEM((NPAD_,), jnp.int32),
            pltpu.VMEM_SHARED((2 * NPAD_,), jnp.int32),
            pltpu.VMEM((40,), jnp.int32),
            pltpu.VMEM((40, D), jnp.float32),
            pltpu.VMEM((40, DA), jnp.float32),
            pltpu.SemaphoreType.DMA,
        ],
    )
    def topk_gather(sims_hbm, vid_hbm, aud_hbm, vout_hbm, aout_hbm,
                    bits_v, hist_v, cum_v, candv_v, candp_v, eq_v, oidx_v,
                    shared, gidx_v, vrows_v, arows_v, sem):
        c = lax.axis_index("c")
        s = lax.axis_index("s")
        row = 2 * c + s
        lane = lax.broadcasted_iota(jnp.int32, (16,), 0)
        ones16 = jnp.full((16,), 1, jnp.int32)
        zeros16 = jnp.zeros((16,), jnp.int32)

        @pl.when(s < 2)
        def _phase_a():
            pltpu.sync_copy(sims_hbm.at[pl.ds(row * T, T)], bits_v)

            def one_pass(shift, prefix, base, first):
                def zero_body(i, _):
                    hist_v[pl.ds(i * 16, 16)] = zeros16
                    return 0
                lax.fori_loop(0, 256, zero_body, 0)

                def scan_body(i, _):
                    for u in range(8):
                        v = bits_v[pl.ds((i * 8 + u) * 16, 16)]
                        d = lax.shift_right_logical(v, shift) & 0xFF
                        if first:
                            m = None
                        else:
                            m = lax.shift_right_logical(v, shift + 8) == prefix
                        plsc.addupdate_scatter(hist_v, [lane * 256 + d],
                                               ones16, mask=m)
                    return 0
                lax.fori_loop(0, T // 128, scan_body, 0)

                def comb_body(i, carry):
                    acc = zeros16
                    for l in range(16):
                        acc = acc + hist_v[pl.ds(l * 256 + i * 16, 16)]
                    cum = plsc.cumsum(acc) + carry
                    cum_v[pl.ds(i * 16, 16)] = cum
                    return carry + jnp.sum(acc)
                lax.fori_loop(0, 16, comb_body, jnp.int32(0))

                def find_body(i, acc):
                    cum = cum_v[pl.ds(i * 16, 16)]
                    below = (base + cum) < K_
                    return acc + jnp.sum(jnp.where(below, 1, 0))
                bucket = lax.fori_loop(0, 16, find_body, jnp.int32(0))

                prev = _scalar(plsc.load_gather(
                    cum_v, [_splat(None, jnp.maximum(bucket - 1, 0))]))
                base2 = base + jnp.where(bucket == 0, 0, prev)
                if first:
                    prefix2 = bucket
                else:
                    prefix2 = (prefix << 8) | bucket
                return prefix2, base2

            prefix, base = one_pass(24, jnp.int32(0), jnp.int32(0), True)
            prefix, base = one_pass(16, prefix, base, False)
            prefix, base = one_pass(8, prefix, base, False)
            kstar, base = one_pass(0, prefix, base, False)
            need = K_ - base

            def coll_body(i, carry):
                off_lt, off_eq = carry
                v = bits_v[pl.ds(i * 16, 16)]
                pos = lane + i * 16
                mlt = v < kstar
                meq = v == kstar
                ilt = jnp.where(mlt, 1, 0)
                ieq = jnp.where(meq, 1, 0)
                dst_lt = off_lt + plsc.cumsum(ilt) - ilt
                dst_eq = off_eq + plsc.cumsum(ieq) - ieq
                plsc.store_scatter(candv_v, [dst_lt], v, mask=mlt)
                plsc.store_scatter(candp_v, [dst_lt], pos, mask=mlt)
                plsc.store_scatter(eq_v, [dst_eq], pos, mask=meq)
                nlt = jnp.sum(ilt)
                neq = jnp.sum(ieq)
                return off_lt + nlt, off_eq + neq
            lax.fori_loop(0, T // 16, coll_body,
                          (jnp.int32(0), jnp.int32(0)))

            def app_body(j, _):
                @pl.when(j * 16 < need)
                def _():
                    candv_v[pl.ds(base + j * 16, 16)] = jnp.full(
                        (16,), kstar, jnp.int32)
                    candp_v[pl.ds(base + j * 16, 16)] = eq_v[pl.ds(j * 16, 16)]
                return 0
            lax.fori_loop(0, 16, app_body, 0)

            def pad_body(m, _):
                oidx_v[pl.ds(m * 16, 16)] = jnp.full((16,), row * T,
                                                     jnp.int32)
                return 0
            lax.fori_loop(0, NPAD_ // 16, pad_body, 0)

            def rank_body(e, _):
                ev = plsc.load_gather(candv_v, [_splat(None, e)])
                ep = plsc.load_gather(candp_v, [_splat(None, e)])
                acc = zeros16
                for i in range(16):
                    v = candv_v[pl.ds(i * 16, 16)]
                    p = candp_v[pl.ds(i * 16, 16)]
                    lt = (v < ev) | ((v == ev) & (p < ep))
                    acc = acc + jnp.where(lt, 1, 0)
                rank = jnp.sum(acc)
                t = _scalar(ep)
                plsc.store_scatter(oidx_v, [_splat(None, rank + 1)],
                                   _splat(None, row * T + t + 1),
                                   mask=lane == 0)
                return 0
            lax.fori_loop(0, K_, rank_body, 0)

            pltpu.sync_copy(oidx_v, shared.at[pl.ds(s * NPAD_, NPAD_)])

        plsc.subcore_barrier()

        br = s // 8
        kk = s % 8
        batch = 2 * c + br
        pltpu.sync_copy(shared.at[pl.ds(br * NPAD_ + 40 * kk, 40)], gidx_v)
        pltpu.async_copy(vid_hbm.at[gidx_v], vrows_v, sem).wait()
        pltpu.sync_copy(vrows_v,
                        vout_hbm.at[pl.ds(batch * NPAD_ + 40 * kk, 40)])
        pltpu.async_copy(aud_hbm.at[gidx_v], arows_v, sem).wait()
        pltpu.sync_copy(arows_v,
                        aout_hbm.at[pl.ds(batch * NPAD_ + 40 * kk, 40)])

    return topk_gather


def kernel(video, audio):
    B, T, D = video.shape
    DA = audio.shape[2]
    sims = _similarities(video)
    sims = sims.at[:, T - 1].set(jnp.inf)
    bits = lax.bitcast_convert_type(sims, jnp.int32)
    vout, aout = _make_topk_gather(B, T, D, DA)(
        bits.reshape(B * T), video.reshape(B * T, D),
        audio.reshape(B * T, DA))
    result_video = vout.reshape(B, NPAD_, D)[:, :K_ + 1]
    result_audio = aout.reshape(B, NPAD_, DA)[:, :K_ + 1]
    return (result_video, result_audio)


# speedup vs baseline: 1.4450x; 1.0613x over previous
import functools

import jax
import jax.numpy as jnp
from jax import lax
from jax.experimental import pallas as pl
from jax.experimental.pallas import tpu as pltpu
from jax.experimental.pallas import tpu_sc as plsc

EPS_ = 1e-05
K_ = 256
BS_ = 4096
NPAD_ = 320


def _chunk_sum(q):
    b = q[:, 0:8]
    for k in range(1, 16):
        b = b + q[:, 8 * k:8 * k + 8]
    e0 = b[:, 0:1] + b[:, 4:5]
    e1 = b[:, 2:3] + b[:, 6:7]
    e2 = b[:, 1:2] + b[:, 5:6]
    e3 = b[:, 3:4] + b[:, 7:8]
    return (e0 + e1) + (e2 + e3)


def _norm2(x):
    s = x * x
    acc = None
    for j in range(3):
        q = s[:, 256 * j:256 * j + 128] + s[:, 256 * j + 128:256 * j + 256]
        c = _chunk_sum(q)
        acc = c if acc is None else acc + c
    return acc


def _rowdot(p):
    acc = None
    for c in range(6):
        sc = _chunk_sum(p[:, 128 * c:128 * c + 128])
        acc = sc if acc is None else acc + sc
    return acc


def _tile_sum(xt):
    b = xt[0:8, :]
    for k in range(1, 16):
        b = b + xt[8 * k:8 * k + 8, :]
    e0 = b[0:1, :] + b[4:5, :]
    e1 = b[2:3, :] + b[6:7, :]
    e2 = b[1:2, :] + b[5:6, :]
    e3 = b[3:4, :] + b[7:8, :]
    return (e0 + e1) + (e2 + e3)


def _norm2_lanes(x):
    s = x * x
    q = [s[:, 256 * j:256 * j + 128] + s[:, 256 * j + 128:256 * j + 256]
         for j in range(3)]
    groups = []
    for g in range(x.shape[0] // 128):
        acc = None
        for j in range(3):
            c = _tile_sum(q[j][128 * g:128 * g + 128, :].T)
            acc = c if acc is None else acc + c
        groups.append(acc)
    return jnp.concatenate(groups, axis=0)


def _rowdot_lanes(p):
    groups = []
    for g in range(p.shape[0] // 128):
        acc = None
        for c in range(6):
            sc = _tile_sum(p[128 * g:128 * g + 128, 128 * c:128 * c + 128].T)
            acc = sc if acc is None else acc + sc
        groups.append(acc)
    return jnp.concatenate(groups, axis=0)


def _sims_body(a_ref, b_ref, o_ref):
    a = a_ref[0]
    b0 = b_ref[0, 0:1]
    n2l = _norm2_lanes(a)
    an = jnp.sqrt(n2l) + EPS_
    an_col = jnp.concatenate(
        [an[g:g + 1, :].T for g in range(BS_ // 128)], axis=0)
    av = a / an_col
    bn = jnp.sqrt(_norm2(b0)) + EPS_
    bv = b0 / bn
    nxt_v = jnp.concatenate([av[1:], bv], axis=0)
    dl = _rowdot_lanes(av * nxt_v)
    s = jnp.abs(dl)
    for g in range(BS_ // 128):
        o_ref[0, 0, 128 * g:128 * (g + 1)] = s[g, :]


def _similarities(video):
    B, T, D = video.shape
    nj = T // BS_
    out = pl.pallas_call(
        _sims_body,
        grid=(B, nj),
        in_specs=[
            pl.BlockSpec((1, BS_, D), lambda b, j: (b, j, 0)),
            pl.BlockSpec((1, 8, D),
                         lambda b, j: (b, jnp.minimum(j + 1, nj - 1) * (BS_ // 8), 0)),
        ],
        out_specs=pl.BlockSpec((1, 1, BS_), lambda b, j: (b * nj + j, 0, 0)),
        out_shape=jax.ShapeDtypeStruct((B * nj, 1, BS_), jnp.float32),
    )(video, video)
    return out.reshape(B, T)


def _splat(x, val, dtype=jnp.int32):
    del x
    return jnp.full((16,), val, dtype)


def _scalar(v16):
    return lax.reduce_max(v16, axes=(0,))


def _make_topk_gather(B, T, D, DA):
    mesh = plsc.VectorSubcoreMesh(core_axis_name="c", subcore_axis_name="s")

    @functools.partial(
        pl.kernel,
        out_type=[jax.ShapeDtypeStruct((B * NPAD_, D), jnp.float32),
                  jax.ShapeDtypeStruct((B * NPAD_, DA), jnp.float32)],
        mesh=mesh,
        compiler_params=pltpu.CompilerParams(needs_layout_passes=False),
        scratch_types=[
            pltpu.VMEM((T,), jnp.int32),
            pltpu.VMEM((4096,), jnp.int32),
            pltpu.VMEM((256,), jnp.int32),
            pltpu.VMEM((272,), jnp.int32),
            pltpu.VMEM((272,), jnp.int32),
            pltpu.VMEM((T + 16,), jnp.int32),
            pltpu.VMEM((NPAD_,), jnp.int32),
            pltpu.VMEM_SHARED((2 * NPAD_,), jnp.int32),
            pltpu.VMEM((40,), jnp.int32),
            pltpu.VMEM((40, D), jnp.float32),
            pltpu.VMEM((40, DA), jnp.float32),
            pltpu.SemaphoreType.DMA,
        ],
    )
    def topk_gather(sims_hbm, vid_hbm, aud_hbm, vout_hbm, aout_hbm,
                    bits_v, hist_v, cum_v, candv_v, candp_v, eq_v, oidx_v,
                    shared, gidx_v, vrows_v, arows_v, sem):
        c = lax.axis_index("c")
        s = lax.axis_index("s")
        row = 2 * c + s
        lane = lax.broadcasted_iota(jnp.int32, (16,), 0)
        ones16 = jnp.full((16,), 1, jnp.int32)
        zeros16 = jnp.zeros((16,), jnp.int32)

        @pl.when(s < 2)
        def _phase_a():
            pltpu.sync_copy(sims_hbm.at[pl.ds(row * T, T)], bits_v)

            def one_pass(shift, prefix, base, first):
                def zero_body(i, _):
                    hist_v[pl.ds(i * 16, 16)] = zeros16
                    return 0
                lax.fori_loop(0, 256, zero_body, 0)

                def scan_body(i, _):
                    for u in range(8):
                        v = bits_v[pl.ds((i * 8 + u) * 16, 16)]
                        d = lax.shift_right_logical(v, shift) & 0xFF
                        if first:
                            m = None
                        else:
                            m = lax.shift_right_logical(v, shift + 8) == prefix
                        plsc.addupdate_scatter(hist_v, [lane * 256 + d],
                                               ones16, mask=m)
                    return 0
                lax.fori_loop(0, T // 128, scan_body, 0)

                def comb_body(i, carry):
                    acc = zeros16
                    for l in range(16):
                        acc = acc + hist_v[pl.ds(l * 256 + i * 16, 16)]
                    cum = plsc.cumsum(acc) + carry
                    cum_v[pl.ds(i * 16, 16)] = cum
                    return carry + jnp.sum(acc)
                lax.fori_loop(0, 16, comb_body, jnp.int32(0))

                def find_body(i, acc):
                    cum = cum_v[pl.ds(i * 16, 16)]
                    below = (base + cum) < K_
                    return acc + jnp.sum(jnp.where(below, 1, 0))
                bucket = lax.fori_loop(0, 16, find_body, jnp.int32(0))

                prev = _scalar(plsc.load_gather(
                    cum_v, [_splat(None, jnp.maximum(bucket - 1, 0))]))
                base2 = base + jnp.where(bucket == 0, 0, prev)
                if first:
                    prefix2 = bucket
                else:
                    prefix2 = (prefix << 8) | bucket
                return prefix2, base2

            prefix, base = one_pass(24, jnp.int32(0), jnp.int32(0), True)
            prefix, base = one_pass(16, prefix, base, False)
            prefix, base = one_pass(8, prefix, base, False)
            kstar, base = one_pass(0, prefix, base, False)
            need = K_ - base

            def coll_body(i, carry):
                off_lt, off_eq = carry
                v = bits_v[pl.ds(i * 16, 16)]
                pos = lane + i * 16
                mlt = v < kstar
                meq = v == kstar
                ilt = jnp.where(mlt, 1, 0)
                ieq = jnp.where(meq, 1, 0)
                dst_lt = off_lt + plsc.cumsum(ilt) - ilt
                dst_eq = off_eq + plsc.cumsum(ieq) - ieq
                plsc.store_scatter(candv_v, [dst_lt], v, mask=mlt)
                plsc.store_scatter(candp_v, [dst_lt], pos, mask=mlt)
                plsc.store_scatter(eq_v, [dst_eq], pos, mask=meq)
                nlt = jnp.sum(ilt)
                neq = jnp.sum(ieq)
                return off_lt + nlt, off_eq + neq
            lax.fori_loop(0, T // 16, coll_body,
                          (jnp.int32(0), jnp.int32(0)))

            def app_body(j, _):
                @pl.when(j * 16 < need)
                def _():
                    candv_v[pl.ds(base + j * 16, 16)] = jnp.full(
                        (16,), kstar, jnp.int32)
                    candp_v[pl.ds(base + j * 16, 16)] = eq_v[pl.ds(j * 16, 16)]
                return 0
            lax.fori_loop(0, 16, app_body, 0)

            def pad_body(m, _):
                oidx_v[pl.ds(m * 16, 16)] = jnp.full((16,), row * T,
                                                     jnp.int32)
                return 0
            lax.fori_loop(0, NPAD_ // 16, pad_body, 0)

            def rank_body(e, _):
                ev = plsc.load_gather(candv_v, [_splat(None, e)])
                ep = plsc.load_gather(candp_v, [_splat(None, e)])
                acc = zeros16
                for i in range(16):
                    v = candv_v[pl.ds(i * 16, 16)]
                    p = candp_v[pl.ds(i * 16, 16)]
                    lt = (v < ev) | ((v == ev) & (p < ep))
                    acc = acc + jnp.where(lt, 1, 0)
                rank = jnp.sum(acc)
                t = _scalar(ep)
                plsc.store_scatter(oidx_v, [_splat(None, rank + 1)],
                                   _splat(None, row * T + t + 1),
                                   mask=lane == 0)
                return 0
            lax.fori_loop(0, K_, rank_body, 0)

            pltpu.sync_copy(oidx_v, shared.at[pl.ds(s * NPAD_, NPAD_)])

        plsc.subcore_barrier()

        br = s // 8
        kk = s % 8
        batch = 2 * c + br
        pltpu.sync_copy(shared.at[pl.ds(br * NPAD_ + 40 * kk, 40)], gidx_v)
        cv = pltpu.async_copy(vid_hbm.at[gidx_v], vrows_v, sem)
        ca = pltpu.async_copy(aud_hbm.at[gidx_v], arows_v, sem)
        cv.wait()
        pltpu.sync_copy(vrows_v,
                        vout_hbm.at[pl.ds(batch * NPAD_ + 40 * kk, 40)])
        ca.wait()
        pltpu.sync_copy(arows_v,
                        aout_hbm.at[pl.ds(batch * NPAD_ + 40 * kk, 40)])

    return topk_gather


def kernel(video, audio):
    B, T, D = video.shape
    DA = audio.shape[2]
    sims = _similarities(video)
    sims = sims.at[:, T - 1].set(jnp.inf)
    bits = lax.bitcast_convert_type(sims, jnp.int32)
    vout, aout = _make_topk_gather(B, T, D, DA)(
        bits.reshape(B * T), video.reshape(B * T, D),
        audio.reshape(B * T, DA))
    result_video = vout.reshape(B, NPAD_, D)[:, :K_ + 1]
    result_audio = aout.reshape(B, NPAD_, DA)[:, :K_ + 1]
    return (result_video, result_audio)
